# baseline (speedup 1.0000x reference)
import functools

import jax
import jax.numpy as jnp
from jax import lax
from jax.experimental import pallas as pl
from jax.experimental.pallas import tpu as pltpu
from jax.experimental.pallas import tpu_sc as plsc

_RING = 4


@functools.lru_cache(maxsize=None)
def _make_gather_kernel(V, D, B):
    info = plsc.get_sparse_core_info()
    NC, NS, L = info.num_cores, info.num_subcores, info.num_lanes
    NW = NC * NS
    assert D == L and B % NW == 0 and (B // NW) % 8 == 0
    b_per_w = B // NW
    n_full = V // 128
    tail = V - n_full * 128
    mesh = plsc.VectorSubcoreMesh(core_axis_name="c", subcore_axis_name="s")

    @functools.partial(
        pl.kernel,
        mesh=mesh,
        out_type=jax.ShapeDtypeStruct((NW * D * b_per_w,), jnp.float32),
        scratch_types=[
            pltpu.VMEM((b_per_w,), jnp.int32),
            pltpu.VMEM((_RING, D, 128), jnp.float32),
            pltpu.VMEM((D * b_per_w,), jnp.float32),
            pltpu.VMEM((max(D * tail, 8),), jnp.float32),
            pltpu.SemaphoreType.DMA,
            *([pltpu.SemaphoreType.DMA] * _RING),
        ],
        compiler_params=pltpu.CompilerParams(needs_layout_passes=False),
    )
    def gather_kernel(
        tabT_hbm, tail_hbm, idx_hbm, out_hbm, idx_v, ring_v, stg_v, tail_v,
        sem_i, *sems
    ):
        wid = lax.axis_index("s") * NC + lax.axis_index("c")
        base = wid * b_per_w
        pltpu.async_copy(idx_hbm.at[pl.ds(base, b_per_w)], idx_v, sem_i).wait()

        def id_splat(i):
            return plsc.load_gather(idx_v, [jnp.broadcast_to(i, (D,))])

        def id_scalar(splat):
            return lax.reduce_max(splat, (0,))
        if tail:
            pltpu.sync_copy(tail_hbm, tail_v)

        def fire(i, j):
            c = id_scalar(id_splat(i)) >> 7

            @pl.when(c < n_full)
            def _():
                start = pl.multiple_of(c * 128, 128)
                pltpu.make_async_copy(
                    tabT_hbm.at[:, pl.ds(start, 128)], ring_v.at[j], sems[j]
                ).start()

        def drain(i, j):
            c = id_scalar(id_splat(i)) >> 7

            @pl.when(c < n_full)
            def _():
                pltpu.make_async_copy(
                    tabT_hbm.at[:, pl.ds(0, 128)], ring_v.at[j], sems[j]
                ).wait()

        def extract(i, j):
            vsplat = id_splat(i)
            c = id_scalar(vsplat) >> 7
            d16 = lax.iota(jnp.int32, D)

            @pl.when(c < n_full)
            def _():
                col = plsc.load_gather(ring_v.at[j], [d16, vsplat & 127])
                plsc.store_scatter(stg_v, [d16 * b_per_w + i], col)

            if tail:
                @pl.when(c >= n_full)
                def _():
                    col = plsc.load_gather(tail_v, [d16 * tail + (vsplat - n_full * 128)])
                    plsc.store_scatter(stg_v, [d16 * b_per_w + i], col)

        def outer(g, carry):
            fire(g, 0)
            drain(g, 0)
            extract(g, 0)
            return carry

        lax.fori_loop(0, b_per_w, outer, 0)
        d16 = lax.iota(jnp.int32, D)
        chk = plsc.load_gather(stg_v, [d16 * b_per_w + (b_per_w - 1)])
        dep = lax.reduce_max(plsc.bitcast(chk, jnp.int32), (0,)) & 0
        pltpu.sync_copy(stg_v, out_hbm.at[pl.ds(base * D + dep, D * b_per_w)])

    return gather_kernel


def kernel(inputs, table):
    V, D = table.shape
    B = inputs.shape[0]
    info = plsc.get_sparse_core_info()
    NW = info.num_cores * info.num_subcores
    b_per_w = B // NW
    n_full = V // 128
    tail = V - n_full * 128
    idx = jnp.reshape(inputs, (-1,)).astype(jnp.int32)
    if tail:
        tailT = jnp.reshape(table[n_full * 128 :, :].T, (-1,))
    else:
        tailT = jnp.zeros((8,), jnp.float32)
    out1 = _make_gather_kernel(V, D, B)(table.T, tailT, idx)
    return out1.reshape(NW, D, b_per_w).transpose(1, 0, 2).reshape(D, B).T


# speedup vs baseline: 3.0930x; 3.0930x over previous
import functools

import jax
import jax.numpy as jnp
from jax import lax
from jax.experimental import pallas as pl
from jax.experimental.pallas import tpu as pltpu
from jax.experimental.pallas import tpu_sc as plsc

_RING = 4


@functools.lru_cache(maxsize=None)
def _make_gather_kernel(V, D, B):
    info = plsc.get_sparse_core_info()
    NC, NS, L = info.num_cores, info.num_subcores, info.num_lanes
    NW = NC * NS
    assert D == L and B % NW == 0 and (B // NW) % 8 == 0
    b_per_w = B // NW
    n_full = V // 128
    tail = V - n_full * 128
    mesh = plsc.VectorSubcoreMesh(core_axis_name="c", subcore_axis_name="s")

    @functools.partial(
        pl.kernel,
        mesh=mesh,
        out_type=jax.ShapeDtypeStruct((NW * D * b_per_w,), jnp.float32),
        scratch_types=[
            pltpu.VMEM((b_per_w,), jnp.int32),
            pltpu.VMEM((_RING, D, 128), jnp.float32),
            pltpu.VMEM((D * b_per_w,), jnp.float32),
            pltpu.VMEM((max(D * tail, 8),), jnp.float32),
            pltpu.SemaphoreType.DMA,
            *([pltpu.SemaphoreType.DMA] * _RING),
        ],
        compiler_params=pltpu.CompilerParams(needs_layout_passes=False),
    )
    def gather_kernel(
        tabT_hbm, tail_hbm, idx_hbm, out_hbm, idx_v, ring_v, stg_v, tail_v,
        sem_i, *sems
    ):
        wid = lax.axis_index("s") * NC + lax.axis_index("c")
        base = wid * b_per_w
        pltpu.async_copy(idx_hbm.at[pl.ds(base, b_per_w)], idx_v, sem_i).wait()

        def id_splat(i):
            return plsc.load_gather(idx_v, [jnp.broadcast_to(i, (D,))])

        def id_scalar(splat):
            return lax.reduce_max(splat, (0,))
        if tail:
            pltpu.sync_copy(tail_hbm, tail_v)

        def fire(i, j):
            c = id_scalar(id_splat(i)) >> 7

            @pl.when(c < n_full)
            def _():
                start = pl.multiple_of(c * 128, 128)
                pltpu.make_async_copy(
                    tabT_hbm.at[:, pl.ds(start, 128)], ring_v.at[j], sems[j]
                ).start()

        def drain(i, j):
            c = id_scalar(id_splat(i)) >> 7

            @pl.when(c < n_full)
            def _():
                pltpu.make_async_copy(
                    tabT_hbm.at[:, pl.ds(0, 128)], ring_v.at[j], sems[j]
                ).wait()

        def extract(i, j):
            vsplat = id_splat(i)
            c = id_scalar(vsplat) >> 7
            d16 = lax.iota(jnp.int32, D)

            @pl.when(c < n_full)
            def _():
                col = plsc.load_gather(ring_v.at[j], [d16, vsplat & 127])
                plsc.store_scatter(stg_v, [d16 * b_per_w + i], col)

            if tail:
                @pl.when(c >= n_full)
                def _():
                    col = plsc.load_gather(tail_v, [d16 * tail + (vsplat - n_full * 128)])
                    plsc.store_scatter(stg_v, [d16 * b_per_w + i], col)

        def outer(g, carry):
            for j in range(_RING):
                i = g * _RING + j

                @pl.when(g > 0)
                def _():
                    drain(i - _RING, j)
                    extract(i - _RING, j)

                @pl.when(i < b_per_w)
                def _():
                    fire(i, j)

            return carry

        lax.fori_loop(0, b_per_w // _RING + 1, outer, 0)
        d16 = lax.iota(jnp.int32, D)
        chk = plsc.load_gather(stg_v, [d16 * b_per_w + (b_per_w - 1)])
        dep = lax.reduce_max(plsc.bitcast(chk, jnp.int32), (0,)) & 0
        pltpu.sync_copy(stg_v, out_hbm.at[pl.ds(base * D + dep, D * b_per_w)])

    return gather_kernel


def kernel(inputs, table):
    V, D = table.shape
    B = inputs.shape[0]
    info = plsc.get_sparse_core_info()
    NW = info.num_cores * info.num_subcores
    b_per_w = B // NW
    n_full = V // 128
    tail = V - n_full * 128
    idx = jnp.reshape(inputs, (-1,)).astype(jnp.int32)
    if tail:
        tailT = jnp.reshape(table[n_full * 128 :, :].T, (-1,))
    else:
        tailT = jnp.zeros((8,), jnp.float32)
    out1 = _make_gather_kernel(V, D, B)(table.T, tailT, idx)
    return out1.reshape(NW, D, b_per_w).transpose(1, 0, 2).reshape(D, B).T


# speedup vs baseline: 3.1090x; 1.0052x over previous
import functools

import jax
import jax.numpy as jnp
from jax import lax
from jax.experimental import pallas as pl
from jax.experimental.pallas import tpu as pltpu
from jax.experimental.pallas import tpu_sc as plsc

_RING = 4


@functools.lru_cache(maxsize=None)
def _make_gather_kernel(V, D, B):
    info = plsc.get_sparse_core_info()
    NC, NS, L = info.num_cores, info.num_subcores, info.num_lanes
    NW = NC * NS
    assert D == L and B % NW == 0 and (B // NW) % 8 == 0
    b_per_w = B // NW
    n_full = V // 128
    tail = V - n_full * 128
    mesh = plsc.VectorSubcoreMesh(core_axis_name="c", subcore_axis_name="s")

    @functools.partial(
        pl.kernel,
        mesh=mesh,
        out_type=jax.ShapeDtypeStruct((NW * D * b_per_w,), jnp.float32),
        scratch_types=[
            pltpu.VMEM((b_per_w + 16,), jnp.int32),
            pltpu.VMEM((_RING, D, 128), jnp.float32),
            pltpu.VMEM((D * b_per_w,), jnp.float32),
            pltpu.VMEM((max(D * tail, 8),), jnp.float32),
            pltpu.SemaphoreType.DMA,
            *([pltpu.SemaphoreType.DMA] * _RING),
        ],
        compiler_params=pltpu.CompilerParams(needs_layout_passes=False),
    )
    def gather_kernel(
        tabT_hbm, tail_hbm, idx_hbm, out_hbm, idx_v, ring_v, stg_v, tail_v,
        sem_i, *sems
    ):
        wid = lax.axis_index("s") * NC + lax.axis_index("c")
        base = wid * b_per_w
        pltpu.async_copy(
            idx_hbm.at[pl.ds(base, b_per_w)], idx_v.at[pl.ds(0, b_per_w)], sem_i
        ).wait()

        def id_at(i):
            return idx_v[pl.ds(i, 16)][0]
        if tail:
            pltpu.sync_copy(tail_hbm, tail_v)

        def fire(i, j):
            c = id_at(i) >> 7

            @pl.when(c < n_full)
            def _():
                start = pl.multiple_of(c * 128, 128)
                pltpu.make_async_copy(
                    tabT_hbm.at[:, pl.ds(start, 128)], ring_v.at[j], sems[j]
                ).start()

        def drain(i, j):
            c = id_at(i) >> 7

            @pl.when(c < n_full)
            def _():
                pltpu.make_async_copy(
                    tabT_hbm.at[:, pl.ds(0, 128)], ring_v.at[j], sems[j]
                ).wait()

        def extract(i, j):
            v = id_at(i)
            c = v >> 7
            vsplat = jnp.broadcast_to(v, (D,))
            d16 = lax.iota(jnp.int32, D)

            @pl.when(c < n_full)
            def _():
                col = plsc.load_gather(ring_v.at[j], [d16, vsplat & 127])
                plsc.store_scatter(stg_v, [d16 * b_per_w + i], col)

            if tail:
                @pl.when(c >= n_full)
                def _():
                    col = plsc.load_gather(tail_v, [d16 * tail + (vsplat - n_full * 128)])
                    plsc.store_scatter(stg_v, [d16 * b_per_w + i], col)

        def outer(g, carry):
            for j in range(_RING):
                i = g * _RING + j

                @pl.when(g > 0)
                def _():
                    drain(i - _RING, j)
                    extract(i - _RING, j)

                @pl.when(i < b_per_w)
                def _():
                    fire(i, j)

            return carry

        lax.fori_loop(0, b_per_w // _RING + 1, outer, 0)
        d16 = lax.iota(jnp.int32, D)
        chk = plsc.load_gather(stg_v, [d16 * b_per_w + (b_per_w - 1)])
        dep = lax.reduce_max(plsc.bitcast(chk, jnp.int32), (0,)) & 0
        pltpu.sync_copy(stg_v, out_hbm.at[pl.ds(base * D + dep, D * b_per_w)])

    return gather_kernel


def kernel(inputs, table):
    V, D = table.shape
    B = inputs.shape[0]
    info = plsc.get_sparse_core_info()
    NW = info.num_cores * info.num_subcores
    b_per_w = B // NW
    n_full = V // 128
    tail = V - n_full * 128
    idx = jnp.reshape(inputs, (-1,)).astype(jnp.int32)
    if tail:
        tailT = jnp.reshape(table[n_full * 128 :, :].T, (-1,))
    else:
        tailT = jnp.zeros((8,), jnp.float32)
    out1 = _make_gather_kernel(V, D, B)(table.T, tailT, idx)
    return out1.reshape(NW, D, b_per_w).transpose(1, 0, 2).reshape(D, B).T


# speedup vs baseline: 4.5326x; 1.4579x over previous
import functools

import jax
import jax.numpy as jnp
from jax import lax
from jax.experimental import pallas as pl
from jax.experimental.pallas import tpu as pltpu
from jax.experimental.pallas import tpu_sc as plsc

_RING = 8


@functools.lru_cache(maxsize=None)
def _make_gather_kernel(V, D, B):
    info = plsc.get_sparse_core_info()
    NC, NS, L = info.num_cores, info.num_subcores, info.num_lanes
    NW = NC * NS
    assert D == L and B % NW == 0 and (B // NW) % 8 == 0
    b_per_w = B // NW
    n_full = V // 128
    tail = V - n_full * 128
    mesh = plsc.VectorSubcoreMesh(core_axis_name="c", subcore_axis_name="s")

    @functools.partial(
        pl.kernel,
        mesh=mesh,
        out_type=jax.ShapeDtypeStruct((NW * D * b_per_w,), jnp.float32),
        scratch_types=[
            pltpu.VMEM((b_per_w + 16,), jnp.int32),
            pltpu.VMEM((_RING, D, 128), jnp.float32),
            pltpu.VMEM((D * b_per_w,), jnp.float32),
            pltpu.VMEM((max(D * tail, 8),), jnp.float32),
            pltpu.SemaphoreType.DMA,
            *([pltpu.SemaphoreType.DMA] * _RING),
        ],
        compiler_params=pltpu.CompilerParams(needs_layout_passes=False),
    )
    def gather_kernel(
        tabT_hbm, tail_hbm, idx_hbm, out_hbm, idx_v, ring_v, stg_v, tail_v,
        sem_i, *sems
    ):
        wid = lax.axis_index("s") * NC + lax.axis_index("c")
        base = wid * b_per_w
        pltpu.async_copy(
            idx_hbm.at[pl.ds(base, b_per_w)], idx_v.at[pl.ds(0, b_per_w)], sem_i
        ).wait()

        def id_at(i):
            return idx_v[pl.ds(i, 16)][0]
        if tail:
            pltpu.sync_copy(tail_hbm, tail_v)

        def fire(i, j):
            c = id_at(i) >> 7

            @pl.when(c < n_full)
            def _():
                start = pl.multiple_of(c * 128, 128)
                pltpu.make_async_copy(
                    tabT_hbm.at[:, pl.ds(start, 128)], ring_v.at[j], sems[j]
                ).start()

        def drain(i, j):
            c = id_at(i) >> 7

            @pl.when(c < n_full)
            def _():
                pltpu.make_async_copy(
                    tabT_hbm.at[:, pl.ds(0, 128)], ring_v.at[j], sems[j]
                ).wait()

        def extract(i, j):
            v = id_at(i)
            c = v >> 7
            vsplat = jnp.broadcast_to(v, (D,))
            d16 = lax.iota(jnp.int32, D)

            @pl.when(c < n_full)
            def _():
                col = plsc.load_gather(ring_v.at[j], [d16, vsplat & 127])
                plsc.store_scatter(stg_v, [d16 * b_per_w + i], col)

            if tail:
                @pl.when(c >= n_full)
                def _():
                    col = plsc.load_gather(tail_v, [d16 * tail + (vsplat - n_full * 128)])
                    plsc.store_scatter(stg_v, [d16 * b_per_w + i], col)

        def outer(g, carry):
            for j in range(_RING):
                i = g * _RING + j

                @pl.when(g > 0)
                def _():
                    drain(i - _RING, j)
                    extract(i - _RING, j)

                @pl.when(i < b_per_w)
                def _():
                    fire(i, j)

            return carry

        lax.fori_loop(0, b_per_w // _RING + 1, outer, 0)
        d16 = lax.iota(jnp.int32, D)
        chk = plsc.load_gather(stg_v, [d16 * b_per_w + (b_per_w - 1)])
        dep = lax.reduce_max(plsc.bitcast(chk, jnp.int32), (0,)) & 0
        pltpu.sync_copy(stg_v, out_hbm.at[pl.ds(base * D + dep, D * b_per_w)])

    return gather_kernel


def kernel(inputs, table):
    V, D = table.shape
    B = inputs.shape[0]
    info = plsc.get_sparse_core_info()
    NW = info.num_cores * info.num_subcores
    b_per_w = B // NW
    n_full = V // 128
    tail = V - n_full * 128
    idx = jnp.reshape(inputs, (-1,)).astype(jnp.int32)
    if tail:
        tailT = jnp.reshape(table[n_full * 128 :, :].T, (-1,))
    else:
        tailT = jnp.zeros((8,), jnp.float32)
    out1 = _make_gather_kernel(V, D, B)(table.T, tailT, idx)
    return out1.reshape(NW, D, b_per_w).transpose(1, 0, 2).reshape(D, B).T
